# baseline (speedup 1.0000x reference)
import functools

import jax
import jax.numpy as jnp
from jax import lax
from jax.experimental import pallas as pl
from jax.experimental.pallas import tpu as pltpu
from jax.experimental.pallas import tpu_sc as plsc

_NU = 1
_L = 16
_NW = 32
_EBIAS = 112 << 11


def _prep_body(nie_ref, dist_ref, idx_ref, x_ref, packed_ref, comb_ref,
               sxx_ref):
    i = pl.program_id(0)
    e = jnp.exp(dist_ref[...] * nie_ref[0, 0])
    eb = e.astype(jnp.bfloat16)
    ones8 = jnp.ones((8, eb.shape[1]), jnp.bfloat16)
    s8 = lax.dot_general(ones8, eb, (((1,), (1,)), ((), ())),
                         preferred_element_type=jnp.float32)
    dinv = 1.0 / s8[0:1, :]
    xb = x_ref[0, :, :]

    def bf16_bits(v):
        u = lax.bitcast_convert_type(v, jnp.uint32)
        r = u + jnp.uint32(0x7FFF) + ((u >> 16) & jnp.uint32(1))
        return r & jnp.uint32(0xFFFF0000)

    packed_ref[0, :, :] = lax.bitcast_convert_type(
        bf16_bits(dinv) | (bf16_bits(xb) >> 16), jnp.int32)

    ebits = lax.bitcast_convert_type(e, jnp.uint32)
    w15 = ((ebits + jnp.uint32(0x800)) >> 12).astype(jnp.int32) - _EBIAS
    w15 = jnp.clip(w15, 0, 0x7FFF).astype(jnp.uint32)
    iw = lax.bitcast_convert_type(idx_ref[...], jnp.uint32)
    comb_ref[...] = lax.bitcast_convert_type((iw << 15) | w15, jnp.int32)

    @pl.when(i == 0)
    def _():
        sxx_ref[0, 0] = 0.0

    sxx_ref[0, 0] += jnp.sum(xb * xb)


def _edge_body(n, kk, comb_hbm, packed_hbm, out_hbm,
               table_v, buf_v, accst_v, sem_c):
    groups = n // _L
    base_g = groups // _NW
    extra = groups % _NW
    gw = _L * kk
    wid = lax.axis_index("c") * _L + lax.axis_index("s")
    g0 = wid * base_g + jnp.minimum(wid, extra)
    ng = base_g + (wid < extra).astype(jnp.int32)

    pltpu.sync_copy(packed_hbm, table_v)

    lanes = jnp.arange(_L, dtype=jnp.int32)
    rowmask = [lanes == r for r in range(_L)]

    def issue(g, p):
        pltpu.async_copy(comb_hbm.at[pl.ds(g * gw, gw)],
                         buf_v.at[pl.ds(p * gw, gw)], sem_c)

    def wait(p):
        pltpu.make_async_copy(comb_hbm.at[pl.ds(0, gw)],
                              buf_v.at[pl.ds(p * gw, gw)], sem_c).wait()

    issue(g0, jnp.int32(0))

    def body(k, acc):
        p = lax.rem(k, 2)
        wait(p)

        @pl.when(k + 1 < ng)
        def _():
            issue(g0 + k + 1, 1 - p)

        g = g0 + k
        rows = g * _L + lanes
        own = plsc.load_gather(table_v, [rows])
        xi = lax.bitcast_convert_type(lax.shift_left(own, 16), jnp.float32)

        den_v = jnp.zeros((_L,), jnp.float32)
        num_v = jnp.zeros((_L,), jnp.float32)
        base = p * gw
        for r in range(_L):
            usum = None
            uxsum = None
            for q in range(kk // _L):
                w = buf_v[pl.ds(base + r * kk + q * _L, _L)]
                idx = lax.shift_right_logical(w, 15)
                ev = lax.bitcast_convert_type(
                    lax.shift_left((w & jnp.int32(0x7FFF)) + _EBIAS, 12),
                    jnp.float32)
                pk = plsc.load_gather(table_v, [idx])
                dg = lax.bitcast_convert_type(pk & jnp.int32(-65536),
                                              jnp.float32)
                xg = lax.bitcast_convert_type(lax.shift_left(pk, 16),
                                              jnp.float32)
                u = ev * dg
                ux = u * xg
                usum = u if usum is None else usum + u
                uxsum = ux if uxsum is None else uxsum + ux
            sden = jnp.sum(usum)
            snum = jnp.sum(uxsum)
            den_v = jnp.where(rowmask[r], sden, den_v)
            num_v = jnp.where(rowmask[r], snum, num_v)
        return acc + xi * num_v / den_v

    acc = lax.fori_loop(0, ng, body, jnp.zeros((_L,), jnp.float32))
    accst_v[...] = acc
    pltpu.sync_copy(accst_v, out_hbm.at[wid])


def kernel(x, indices, distances, eps, k_param):
    n, kk = distances.shape
    nie = (-1.0 / eps).astype(jnp.float32)
    idx32 = indices.astype(jnp.int32)

    bp = 2000
    packed, comb, sxx = pl.pallas_call(
        _prep_body,
        grid=(n // bp,),
        in_specs=[
            pl.BlockSpec(memory_space=pltpu.SMEM),
            pl.BlockSpec((bp, kk), lambda i: (i, 0)),
            pl.BlockSpec((bp, kk), lambda i: (i, 0)),
            pl.BlockSpec((1, 1, bp), lambda i: (i, 0, 0)),
        ],
        out_specs=[
            pl.BlockSpec((1, 1, bp), lambda i: (i, 0, 0)),
            pl.BlockSpec((bp, kk), lambda i: (i, 0)),
            pl.BlockSpec(memory_space=pltpu.SMEM),
        ],
        out_shape=[
            jax.ShapeDtypeStruct((n // bp, 1, bp), jnp.int32),
            jax.ShapeDtypeStruct((n, kk), jnp.int32),
            jax.ShapeDtypeStruct((1, 1), jnp.float32),
        ],
    )(nie.reshape(1, 1), distances, idx32, x.reshape(n // bp, 1, bp))
    packed = packed.reshape(n)
    comb = comb.reshape(n * kk)

    mesh = plsc.VectorSubcoreMesh(core_axis_name="c", subcore_axis_name="s")
    edge = functools.partial(
        pl.kernel,
        mesh=mesh,
        compiler_params=pltpu.CompilerParams(needs_layout_passes=False),
        out_type=jax.ShapeDtypeStruct((_NW, _L), jnp.float32),
        scratch_types=[
            pltpu.VMEM((n,), jnp.int32),
            pltpu.VMEM((2 * _L * kk,), jnp.int32),
            pltpu.VMEM((_L,), jnp.float32),
            pltpu.SemaphoreType.DMA,
        ],
    )(functools.partial(_edge_body, n, kk))
    parts = edge(comb, packed)

    c0 = 4.0 / eps + 2.0 * _NU / (k_param * k_param) + 10.0
    out = c0 * sxx[0, 0] - (4.0 / eps) * jnp.sum(parts)
    return out.astype(jnp.float32)
